# baseline (speedup 1.0000x reference)
import functools

import jax
import jax.numpy as jnp
from jax import lax
from jax.experimental import pallas as pl
from jax.experimental.pallas import tpu as pltpu
from jax.experimental.pallas import tpu_sc as plsc

T = 2048
D = 768
F = 3072
E = 8
B = 1024
NBLK = (2 * T) // B + (E - 1)
PAD = NBLK * B
NW = 32
TPW = T // NW


def _router_body(x_ref, gw_ref, gb_ref,
                 pos0_ref, pos1_ref, w0_ref, w1_ref, be_ref, tot_ref):
    x = x_ref[...]
    gw = gw_ref[...]
    gb = gb_ref[...]
    l = lax.dot_general(x, gw, (((1,), (1,)), ((), ())),
                        preferred_element_type=jnp.float32) + gb
    ie = lax.broadcasted_iota(jnp.int32, (T, E), 1)
    m1 = jnp.max(l, axis=1, keepdims=True)
    e1 = jnp.min(jnp.where(l == m1, ie, E), axis=1, keepdims=True)
    oh1 = ie == e1
    l2 = jnp.where(oh1, -1e30, l)
    m2 = jnp.max(l2, axis=1, keepdims=True)
    e2 = jnp.min(jnp.where(l2 == m2, ie, E), axis=1, keepdims=True)
    oh2 = ie == e2
    s = jnp.exp(m2 - m1)
    w0 = 1.0 / (1.0 + s)
    w1 = s * w0
    oh1f = oh1.astype(jnp.float32)
    oh2f = oh2.astype(jnp.float32)
    C = 256
    ric = lax.broadcasted_iota(jnp.int32, (C, C), 0)
    cic = lax.broadcasted_iota(jnp.int32, (C, C), 1)
    tric = (cic < ric).astype(jnp.bfloat16)
    oh1b = oh1.astype(jnp.bfloat16)
    oh2b = oh2.astype(jnp.bfloat16)
    r1_chunks = []
    r2_chunks = []
    off1 = jnp.zeros((1, E), jnp.float32)
    off2 = jnp.zeros((1, E), jnp.float32)
    for k in range(T // C):
        sl = slice(k * C, (k + 1) * C)
        w1c = lax.dot_general(tric, oh1b[sl], (((1,), (0,)), ((), ())),
                              preferred_element_type=jnp.float32)
        w2c = lax.dot_general(tric, oh2b[sl], (((1,), (0,)), ((), ())),
                              preferred_element_type=jnp.float32)
        r1_chunks.append(w1c + off1)
        r2_chunks.append(w2c + off2)
        off1 = off1 + jnp.sum(oh1f[sl], axis=0, keepdims=True)
        off2 = off2 + jnp.sum(oh2f[sl], axis=0, keepdims=True)
    r1 = jnp.concatenate(r1_chunks, axis=0)
    r2 = jnp.concatenate(r2_chunks, axis=0)
    c1 = off1
    c2 = off2
    cnt = c1 + c2
    nb = jnp.floor((cnt + (B - 1)) * (1.0 / B))
    re_ = lax.broadcasted_iota(jnp.int32, (E, E), 0)
    ce_ = lax.broadcasted_iota(jnp.int32, (E, E), 1)
    m8 = (re_ < ce_).astype(jnp.float32)
    pblk = lax.dot_general(nb, m8, (((1,), (0,)), ((), ())),
                           preferred_element_type=jnp.float32)
    P = pblk * B
    pos0 = jnp.sum(oh1f * (P + r1), axis=1, keepdims=True)
    pos1 = jnp.sum(oh2f * (P + c1 + r2), axis=1, keepdims=True)
    pos0_ref[...] = pos0.astype(jnp.int32)
    pos1_ref[...] = pos1.astype(jnp.int32)
    w0_ref[...] = jnp.broadcast_to(w0, (T, 16))
    w1_ref[...] = jnp.broadcast_to(w1, (T, 16))
    tot = jnp.sum(nb)
    bi = lax.broadcasted_iota(jnp.int32, (NBLK, E), 0).astype(jnp.float32)
    bi = jnp.minimum(bi, tot - 1.0)
    ind = ((bi >= P * (1.0 / B)) & (bi < P * (1.0 / B) + nb)).astype(jnp.float32)
    ecol = lax.broadcasted_iota(jnp.int32, (NBLK, E), 1).astype(jnp.float32)
    be_ref[...] = jnp.sum(ind * ecol, axis=1, keepdims=True).astype(jnp.int32)
    tot_ref[...] = jnp.reshape(tot, (1, 1)).astype(jnp.int32)


def _ffn_body(be_ref, tot_ref, xb_ref, w1_ref, b1_ref, w2_ref, b2_ref, out_ref):
    i = pl.program_id(0)

    @pl.when(i < tot_ref[0])
    def _():
        xb = xb_ref[...].astype(jnp.bfloat16)
        w1b16 = w1_ref[0].astype(jnp.bfloat16)
        h = lax.dot_general(xb, w1b16, (((1,), (1,)), ((), ())),
                            preferred_element_type=jnp.float32) + b1_ref[0]
        h = 0.5 * h * (1.0 + lax.erf(h * 0.7071067811865476))
        w2b16 = w2_ref[0].astype(jnp.bfloat16)
        o = lax.dot_general(h.astype(jnp.bfloat16), w2b16,
                            (((1,), (1,)), ((), ())),
                            preferred_element_type=jnp.float32) + b2_ref[0]
        out_ref[...] = o


def _router(x2d, gate_W, gb2):
    return pl.pallas_call(
        _router_body,
        out_shape=[
            jax.ShapeDtypeStruct((T, 1), jnp.int32),
            jax.ShapeDtypeStruct((T, 1), jnp.int32),
            jax.ShapeDtypeStruct((T, 16), jnp.float32),
            jax.ShapeDtypeStruct((T, 16), jnp.float32),
            jax.ShapeDtypeStruct((NBLK, 1), jnp.int32),
            jax.ShapeDtypeStruct((1, 1), jnp.int32),
        ],
    )(x2d, gate_W, gb2)


def _ffn(be, tot, gx, W1, b1, W2, b2):
    grid_spec = pltpu.PrefetchScalarGridSpec(
        num_scalar_prefetch=2,
        grid=(NBLK,),
        in_specs=[
            pl.BlockSpec((B, D), lambda i, be, tot: (i, 0)),
            pl.BlockSpec((1, F, D), lambda i, be, tot: (be[i], 0, 0)),
            pl.BlockSpec((1, 1, F), lambda i, be, tot: (be[i], 0, 0)),
            pl.BlockSpec((1, D, F), lambda i, be, tot: (be[i], 0, 0)),
            pl.BlockSpec((1, 1, D), lambda i, be, tot: (be[i], 0, 0)),
        ],
        out_specs=pl.BlockSpec((B, D), lambda i, be, tot: (i, 0)),
    )
    return pl.pallas_call(
        _ffn_body,
        grid_spec=grid_spec,
        out_shape=jax.ShapeDtypeStruct((PAD, D), jnp.float32),
        compiler_params=pltpu.CompilerParams(
            dimension_semantics=("arbitrary",)),
    )(be, tot, gx, W1, b1.reshape(E, 1, F), W2, b2.reshape(E, 1, D))


@functools.lru_cache(maxsize=1)
def _sc_kernels():
    mesh = plsc.VectorSubcoreMesh(core_axis_name="c", subcore_axis_name="s")

    @functools.partial(
        pl.kernel,
        mesh=mesh,
        out_type=jax.ShapeDtypeStruct((PAD, D), jnp.float32),
        scratch_types=[
            pltpu.VMEM((TPW, D), jnp.float32),
            pltpu.VMEM((TPW,), jnp.int32),
            pltpu.VMEM((TPW,), jnp.int32),
            pltpu.SemaphoreType.DMA,
        ],
    )
    def _dispatch(x_hbm, pos0_hbm, pos1_hbm, gx_hbm, xbuf, idx0, idx1, sem):
        wid = lax.axis_index("s") * 2 + lax.axis_index("c")
        base = wid * TPW
        pltpu.sync_copy(x_hbm.at[pl.ds(base, TPW)], xbuf)
        pltpu.sync_copy(pos0_hbm.at[pl.ds(base, TPW)], idx0)
        pltpu.sync_copy(pos1_hbm.at[pl.ds(base, TPW)], idx1)
        pltpu.async_copy(xbuf, gx_hbm.at[idx0], sem).wait()
        pltpu.async_copy(xbuf, gx_hbm.at[idx1], sem).wait()

    @functools.partial(
        pl.kernel,
        mesh=mesh,
        out_type=jax.ShapeDtypeStruct((T, D), jnp.float32),
        scratch_types=[
            pltpu.VMEM((TPW, D), jnp.float32),
            pltpu.VMEM((TPW, D), jnp.float32),
            pltpu.VMEM((TPW,), jnp.int32),
            pltpu.VMEM((TPW,), jnp.int32),
            pltpu.VMEM((TPW, 16), jnp.float32),
            pltpu.VMEM((TPW, 16), jnp.float32),
            pltpu.SemaphoreType.DMA,
        ],
    )
    def _combine(go_hbm, pos0_hbm, pos1_hbm, w0_hbm, w1_hbm, out_hbm,
                 r0, r1, idx0, idx1, w0v, w1v, sem):
        wid = lax.axis_index("s") * 2 + lax.axis_index("c")
        base = wid * TPW
        pltpu.sync_copy(pos0_hbm.at[pl.ds(base, TPW)], idx0)
        pltpu.sync_copy(pos1_hbm.at[pl.ds(base, TPW)], idx1)
        pltpu.sync_copy(w0_hbm.at[pl.ds(base, TPW)], w0v)
        pltpu.sync_copy(w1_hbm.at[pl.ds(base, TPW)], w1v)
        pltpu.async_copy(go_hbm.at[idx0], r0, sem).wait()
        pltpu.async_copy(go_hbm.at[idx1], r1, sem).wait()

        def body(t, carry):
            a = w0v[t]
            b = w1v[t]
            for j in range(D // 16):
                sl = pl.ds(j * 16, 16)
                r0[t, sl] = r0[t, sl] * a + r1[t, sl] * b
            return carry

        lax.fori_loop(0, TPW, body, 0)
        pltpu.sync_copy(r0, out_hbm.at[pl.ds(base, TPW)])

    return _dispatch, _combine


def kernel(x, gate_W, gate_b, W1, b1, W2, b2):
    x2d = x.reshape(T, D)
    gb2 = gate_b.reshape(1, E)
    pos0c, pos1c, w0b, w1b, be2, tot2 = _router(x2d, gate_W, gb2)
    pos0 = pos0c.reshape(T)
    pos1 = pos1c.reshape(T)
    be = be2.reshape(NBLK)
    tot = tot2.reshape(1)
    dispatch, combine = _sc_kernels()
    gx = dispatch(x2d, pos0, pos1)
    go = _ffn(be, tot, gx, W1, b1, W2, b2)
    out = combine(go, pos0, pos1, w0b, w1b)
    return out.reshape(1, T, D)


# speedup vs baseline: 1.0971x; 1.0971x over previous
import functools

import jax
import jax.numpy as jnp
from jax import lax
from jax.experimental import pallas as pl
from jax.experimental.pallas import tpu as pltpu
from jax.experimental.pallas import tpu_sc as plsc

T = 2048
D = 768
F = 3072
E = 8
B = 512
NBLK = (2 * T) // B + (E - 1)
PAD = NBLK * B
NW = 32
TPW = T // NW


def _router_body(x_ref, gw_ref, gb_ref,
                 pos0_ref, pos1_ref, w0_ref, w1_ref, be_ref, tot_ref):
    x = x_ref[...]
    gw = gw_ref[...]
    gb = gb_ref[...]
    l = lax.dot_general(x, gw, (((1,), (1,)), ((), ())),
                        preferred_element_type=jnp.float32) + gb
    ie = lax.broadcasted_iota(jnp.int32, (T, E), 1)
    m1 = jnp.max(l, axis=1, keepdims=True)
    e1 = jnp.min(jnp.where(l == m1, ie, E), axis=1, keepdims=True)
    oh1 = ie == e1
    l2 = jnp.where(oh1, -1e30, l)
    m2 = jnp.max(l2, axis=1, keepdims=True)
    e2 = jnp.min(jnp.where(l2 == m2, ie, E), axis=1, keepdims=True)
    oh2 = ie == e2
    s = jnp.exp(m2 - m1)
    w0 = 1.0 / (1.0 + s)
    w1 = s * w0
    oh1f = oh1.astype(jnp.float32)
    oh2f = oh2.astype(jnp.float32)
    C = 256
    ric = lax.broadcasted_iota(jnp.int32, (C, C), 0)
    cic = lax.broadcasted_iota(jnp.int32, (C, C), 1)
    tric = (cic < ric).astype(jnp.bfloat16)
    oh1b = oh1.astype(jnp.bfloat16)
    oh2b = oh2.astype(jnp.bfloat16)
    r1_chunks = []
    r2_chunks = []
    off1 = jnp.zeros((1, E), jnp.float32)
    off2 = jnp.zeros((1, E), jnp.float32)
    for k in range(T // C):
        sl = slice(k * C, (k + 1) * C)
        w1c = lax.dot_general(tric, oh1b[sl], (((1,), (0,)), ((), ())),
                              preferred_element_type=jnp.float32)
        w2c = lax.dot_general(tric, oh2b[sl], (((1,), (0,)), ((), ())),
                              preferred_element_type=jnp.float32)
        r1_chunks.append(w1c + off1)
        r2_chunks.append(w2c + off2)
        off1 = off1 + jnp.sum(oh1f[sl], axis=0, keepdims=True)
        off2 = off2 + jnp.sum(oh2f[sl], axis=0, keepdims=True)
    r1 = jnp.concatenate(r1_chunks, axis=0)
    r2 = jnp.concatenate(r2_chunks, axis=0)
    c1 = off1
    c2 = off2
    cnt = c1 + c2
    nb = jnp.floor((cnt + (B - 1)) * (1.0 / B))
    re_ = lax.broadcasted_iota(jnp.int32, (E, E), 0)
    ce_ = lax.broadcasted_iota(jnp.int32, (E, E), 1)
    m8 = (re_ < ce_).astype(jnp.float32)
    pblk = lax.dot_general(nb, m8, (((1,), (0,)), ((), ())),
                           preferred_element_type=jnp.float32)
    P = pblk * B
    pos0 = jnp.sum(oh1f * (P + r1), axis=1, keepdims=True)
    pos1 = jnp.sum(oh2f * (P + c1 + r2), axis=1, keepdims=True)
    pos0_ref[...] = pos0.astype(jnp.int32)
    pos1_ref[...] = pos1.astype(jnp.int32)
    w0_ref[...] = jnp.broadcast_to(w0, (T, 16))
    w1_ref[...] = jnp.broadcast_to(w1, (T, 16))
    tot = jnp.sum(nb)
    bi = lax.broadcasted_iota(jnp.int32, (NBLK, E), 0).astype(jnp.float32)
    bi = jnp.minimum(bi, tot - 1.0)
    ind = ((bi >= P * (1.0 / B)) & (bi < P * (1.0 / B) + nb)).astype(jnp.float32)
    ecol = lax.broadcasted_iota(jnp.int32, (NBLK, E), 1).astype(jnp.float32)
    be_ref[...] = jnp.sum(ind * ecol, axis=1, keepdims=True).astype(jnp.int32)
    tot_ref[...] = jnp.reshape(tot, (1, 1)).astype(jnp.int32)


def _ffn_body(be_ref, tot_ref, xb_ref, w1_ref, b1_ref, w2_ref, b2_ref, out_ref):
    i = pl.program_id(0)

    @pl.when(i < tot_ref[0])
    def _():
        xb = xb_ref[...].astype(jnp.bfloat16)
        w1b16 = w1_ref[0].astype(jnp.bfloat16)
        h = lax.dot_general(xb, w1b16, (((1,), (1,)), ((), ())),
                            preferred_element_type=jnp.float32) + b1_ref[0]
        h = 0.5 * h * (1.0 + lax.erf(h * 0.7071067811865476))
        w2b16 = w2_ref[0].astype(jnp.bfloat16)
        o = lax.dot_general(h.astype(jnp.bfloat16), w2b16,
                            (((1,), (1,)), ((), ())),
                            preferred_element_type=jnp.float32) + b2_ref[0]
        out_ref[...] = o


def _router(x2d, gate_W, gb2):
    return pl.pallas_call(
        _router_body,
        out_shape=[
            jax.ShapeDtypeStruct((T, 1), jnp.int32),
            jax.ShapeDtypeStruct((T, 1), jnp.int32),
            jax.ShapeDtypeStruct((T, 16), jnp.float32),
            jax.ShapeDtypeStruct((T, 16), jnp.float32),
            jax.ShapeDtypeStruct((NBLK, 1), jnp.int32),
            jax.ShapeDtypeStruct((1, 1), jnp.int32),
        ],
    )(x2d, gate_W, gb2)


def _ffn(be, tot, gx, W1, b1, W2, b2):
    grid_spec = pltpu.PrefetchScalarGridSpec(
        num_scalar_prefetch=2,
        grid=(NBLK,),
        in_specs=[
            pl.BlockSpec((B, D), lambda i, be, tot: (i, 0)),
            pl.BlockSpec((1, F, D), lambda i, be, tot: (be[i], 0, 0)),
            pl.BlockSpec((1, 1, F), lambda i, be, tot: (be[i], 0, 0)),
            pl.BlockSpec((1, D, F), lambda i, be, tot: (be[i], 0, 0)),
            pl.BlockSpec((1, 1, D), lambda i, be, tot: (be[i], 0, 0)),
        ],
        out_specs=pl.BlockSpec((B, D), lambda i, be, tot: (i, 0)),
    )
    return pl.pallas_call(
        _ffn_body,
        grid_spec=grid_spec,
        out_shape=jax.ShapeDtypeStruct((PAD, D), jnp.float32),
        compiler_params=pltpu.CompilerParams(
            dimension_semantics=("arbitrary",)),
    )(be, tot, gx, W1, b1.reshape(E, 1, F), W2, b2.reshape(E, 1, D))


@functools.lru_cache(maxsize=1)
def _sc_kernels():
    mesh = plsc.VectorSubcoreMesh(core_axis_name="c", subcore_axis_name="s")

    @functools.partial(
        pl.kernel,
        mesh=mesh,
        out_type=jax.ShapeDtypeStruct((PAD, D), jnp.float32),
        scratch_types=[
            pltpu.VMEM((TPW, D), jnp.float32),
            pltpu.VMEM((TPW,), jnp.int32),
            pltpu.VMEM((TPW,), jnp.int32),
            pltpu.SemaphoreType.DMA,
        ],
    )
    def _dispatch(x_hbm, pos0_hbm, pos1_hbm, gx_hbm, xbuf, idx0, idx1, sem):
        wid = lax.axis_index("s") * 2 + lax.axis_index("c")
        base = wid * TPW
        pltpu.sync_copy(x_hbm.at[pl.ds(base, TPW)], xbuf)
        pltpu.sync_copy(pos0_hbm.at[pl.ds(base, TPW)], idx0)
        pltpu.sync_copy(pos1_hbm.at[pl.ds(base, TPW)], idx1)
        pltpu.async_copy(xbuf, gx_hbm.at[idx0], sem).wait()
        pltpu.async_copy(xbuf, gx_hbm.at[idx1], sem).wait()

    @functools.partial(
        pl.kernel,
        mesh=mesh,
        out_type=jax.ShapeDtypeStruct((T, D), jnp.float32),
        scratch_types=[
            pltpu.VMEM((TPW, D), jnp.float32),
            pltpu.VMEM((TPW, D), jnp.float32),
            pltpu.VMEM((TPW,), jnp.int32),
            pltpu.VMEM((TPW,), jnp.int32),
            pltpu.VMEM((TPW, 16), jnp.float32),
            pltpu.VMEM((TPW, 16), jnp.float32),
            pltpu.SemaphoreType.DMA,
        ],
    )
    def _combine(go_hbm, pos0_hbm, pos1_hbm, w0_hbm, w1_hbm, out_hbm,
                 r0, r1, idx0, idx1, w0v, w1v, sem):
        wid = lax.axis_index("s") * 2 + lax.axis_index("c")
        base = wid * TPW
        pltpu.sync_copy(pos0_hbm.at[pl.ds(base, TPW)], idx0)
        pltpu.sync_copy(pos1_hbm.at[pl.ds(base, TPW)], idx1)
        pltpu.sync_copy(w0_hbm.at[pl.ds(base, TPW)], w0v)
        pltpu.sync_copy(w1_hbm.at[pl.ds(base, TPW)], w1v)
        pltpu.async_copy(go_hbm.at[idx0], r0, sem).wait()
        pltpu.async_copy(go_hbm.at[idx1], r1, sem).wait()

        def body(t, carry):
            a = w0v[t]
            b = w1v[t]
            for j in range(D // 16):
                sl = pl.ds(j * 16, 16)
                r0[t, sl] = r0[t, sl] * a + r1[t, sl] * b
            return carry

        lax.fori_loop(0, TPW, body, 0)
        pltpu.sync_copy(r0, out_hbm.at[pl.ds(base, TPW)])

    return _dispatch, _combine


def kernel(x, gate_W, gate_b, W1, b1, W2, b2):
    x2d = x.reshape(T, D)
    gb2 = gate_b.reshape(1, E)
    pos0c, pos1c, w0b, w1b, be2, tot2 = _router(x2d, gate_W, gb2)
    pos0 = pos0c.reshape(T)
    pos1 = pos1c.reshape(T)
    be = be2.reshape(NBLK)
    tot = tot2.reshape(1)
    dispatch, combine = _sc_kernels()
    gx = dispatch(x2d, pos0, pos1)
    go = _ffn(be, tot, gx, W1, b1, W2, b2)
    out = combine(go, pos0, pos1, w0b, w1b)
    return out.reshape(1, T, D)
